# baseline (speedup 1.0000x reference)
import functools

import jax
import jax.numpy as jnp
from jax import lax
from jax.experimental import pallas as pl
from jax.experimental.pallas import tpu as pltpu
from jax.experimental.pallas import tpu_sc as plsc

B = 4096
L = 200
E = 64
V = 1000000
OUT = 2
LANES = 16
CH = 8

NC, NS = 2, 16
NW = NC * NS
EPW = B // NW
CE = 16
CT = CE * L
GL = 128
NG = CT // GL
NCHUNK = EPW // CE
BN = 65536
NB = pl.cdiv(V, BN)
VP = NB * BN
SUB = BN // 16
SSH = SUB.bit_length() - 1


def _tc_proj(w_scaled, t_t):
    def body(w_ref, t_ref, o_ref):
        pt = lax.dot_general(
            w_ref[...], t_ref[...], (((1,), (0,)), ((), ())),
            preferred_element_type=jnp.float32,
        )
        stacked = jnp.concatenate(
            [pt[:, j * SUB:(j + 1) * SUB] for j in range(16)], axis=0
        )
        o_ref[...] = jnp.transpose(stacked)

    return pl.pallas_call(
        body,
        grid=(NB,),
        in_specs=[
            pl.BlockSpec((CH, E), lambda j: (0, 0)),
            pl.BlockSpec((E, BN), lambda j: (0, j)),
        ],
        out_specs=pl.BlockSpec((SUB, 128), lambda j: (j, 0)),
        out_shape=jax.ShapeDtypeStruct((VP // 16, 128), jnp.float32),
        compiler_params=pltpu.CompilerParams(
            fuse_transposed_lhs_in_matmul=True,
        ),
    )(w_scaled, t_t)


def _sc_pool(x2, proj):
    mesh = plsc.VectorSubcoreMesh(core_axis_name="c", subcore_axis_name="s")

    @functools.partial(
        pl.kernel,
        out_type=jax.ShapeDtypeStruct((B, LANES), jnp.float32),
        mesh=mesh,
        scratch_types=[
            pltpu.VMEM((4, NG, GL), jnp.int32),
            pltpu.VMEM((2, CT, CH), jnp.float32),
            pltpu.VMEM((EPW, LANES), jnp.float32),
            pltpu.SemaphoreType.DMA,
            pltpu.SemaphoreType.DMA,
            pltpu.SemaphoreType.DMA,
            pltpu.SemaphoreType.DMA,
        ],
        compiler_params=pltpu.CompilerParams(
            use_tc_tiling_on_sc=False, needs_layout_passes=False,
        ),
    )
    def k(x_hbm, p_hbm, out_hbm, idx_v, rows_v, out_v, sem0, sem1,
          isem0, isem1):
        wid = lax.axis_index("s") * NC + lax.axis_index("c")
        sems = (sem0, sem1)
        isems = (isem0, isem1)
        lane = lax.iota(jnp.int32, LANES)
        rowpat = lane >> 3
        colpat = lane & 7

        def idx_copy(c, fire):
            grow0 = (wid * NCHUNK + c) * NG
            return (pltpu.async_copy if fire else pltpu.make_async_copy)(
                x_hbm.at[pl.ds(grow0, NG)], idx_v.at[c % 4], isems[c % 2]
            )

        def gathers(c, fire):
            cps = []
            for j in range(NG):
                cp = (pltpu.async_copy if fire else pltpu.make_async_copy)(
                    p_hbm.at[idx_v.at[c % 4].at[j]],
                    rows_v.at[c % 2].at[pl.ds(j * GL, GL)],
                    sems[c % 2],
                )
                cps.append(cp)
            return cps

        idx_copy(0, True).wait()
        idx_copy(1, True)
        gathers(0, True)
        for c in range(NCHUNK):
            buf = c % 2
            if c + 1 < NCHUNK:
                idx_copy(c + 1, False).wait()
                if c + 2 < NCHUNK:
                    idx_copy(c + 2, True)
                gathers(c + 1, True)
            for cp in gathers(c, False):
                cp.wait()

            rbuf = rows_v.at[buf]

            def elem_body(e, _):
                row0 = e * L

                def t_body(i, accs):
                    a0, a1 = accs
                    r = row0 + i * 20
                    for j in range(5):
                        a0 = a0 + plsc.load_gather(
                            rbuf, [rowpat + (r + 4 * j), colpat])
                        a1 = a1 + plsc.load_gather(
                            rbuf, [rowpat + (r + 4 * j + 2), colpat])
                    return (a0, a1)

                z = jnp.zeros((LANES,), jnp.float32)
                a0, a1 = lax.fori_loop(0, L // 20, t_body, (z, z))
                out_v[c * CE + e, :] = a0 + a1
                return 0

            lax.fori_loop(0, CE, elem_body, 0)

        pltpu.sync_copy(out_v, out_hbm.at[pl.ds(wid * EPW, EPW)])

    return k(x2, proj)


@jax.jit
def _impl(x, table, W, b):
    xi = x.astype(jnp.int32)
    xs = (xi & ~(BN - 1)) | ((xi & (SUB - 1)) << 4) | ((xi & (BN - 1)) >> SSH)
    x2 = xs.reshape(B * L // GL, GL)
    w_scaled = (
        jnp.zeros((CH, E), jnp.float32).at[:OUT, :].set(W) * (1.0 / L)
    )
    proj16 = _tc_proj(w_scaled, table.T)
    proj = proj16.reshape(VP, CH)
    out16 = _sc_pool(x2, proj)
    return out16[:, :OUT] + out16[:, CH:CH + OUT] + b[None, :]


def kernel(x, table, W, b):
    return _impl(x, table, W, b)


# speedup vs baseline: 1.0090x; 1.0090x over previous
import functools

import jax
import jax.numpy as jnp
from jax import lax
from jax.experimental import pallas as pl
from jax.experimental.pallas import tpu as pltpu
from jax.experimental.pallas import tpu_sc as plsc

B = 4096
L = 200
E = 64
V = 1000000
OUT = 2
LANES = 16
CH = 8

NC, NS = 2, 16
NW = NC * NS
EPW = B // NW
CE = 16
CT = CE * L
GL = 128
NG = CT // GL
NCHUNK = EPW // CE
BN = 32768
NB = pl.cdiv(V, BN)
VP = NB * BN
SUB = BN // 16
SSH = SUB.bit_length() - 1


def _tc_proj(w_scaled, t_t):
    def body(w_ref, t_ref, o_ref):
        pt = lax.dot_general(
            w_ref[...], t_ref[...], (((1,), (0,)), ((), ())),
            preferred_element_type=jnp.float32,
        )
        stacked = jnp.concatenate(
            [pt[:, j * SUB:(j + 1) * SUB] for j in range(16)], axis=0
        )
        o_ref[...] = jnp.transpose(stacked)

    return pl.pallas_call(
        body,
        grid=(NB,),
        in_specs=[
            pl.BlockSpec((CH, E), lambda j: (0, 0)),
            pl.BlockSpec((E, BN), lambda j: (0, j)),
        ],
        out_specs=pl.BlockSpec((SUB, 128), lambda j: (j, 0)),
        out_shape=jax.ShapeDtypeStruct((VP // 16, 128), jnp.float32),
        compiler_params=pltpu.CompilerParams(
            fuse_transposed_lhs_in_matmul=True,
        ),
    )(w_scaled, t_t)


def _sc_pool(x2, proj):
    mesh = plsc.VectorSubcoreMesh(core_axis_name="c", subcore_axis_name="s")

    @functools.partial(
        pl.kernel,
        out_type=jax.ShapeDtypeStruct((B, LANES), jnp.float32),
        mesh=mesh,
        scratch_types=[
            pltpu.VMEM((4, NG, GL), jnp.int32),
            pltpu.VMEM((2, CT, CH), jnp.float32),
            pltpu.VMEM((EPW, LANES), jnp.float32),
            pltpu.SemaphoreType.DMA,
            pltpu.SemaphoreType.DMA,
            pltpu.SemaphoreType.DMA,
            pltpu.SemaphoreType.DMA,
        ],
        compiler_params=pltpu.CompilerParams(
            use_tc_tiling_on_sc=False, needs_layout_passes=False,
        ),
    )
    def k(x_hbm, p_hbm, out_hbm, idx_v, rows_v, out_v, sem0, sem1,
          isem0, isem1):
        wid = lax.axis_index("s") * NC + lax.axis_index("c")
        sems = (sem0, sem1)
        isems = (isem0, isem1)
        lane = lax.iota(jnp.int32, LANES)
        rowpat = lane >> 3
        colpat = lane & 7

        def idx_copy(c, fire):
            grow0 = (wid * NCHUNK + c) * NG
            return (pltpu.async_copy if fire else pltpu.make_async_copy)(
                x_hbm.at[pl.ds(grow0, NG)], idx_v.at[c % 4], isems[c % 2]
            )

        def gathers(c, fire):
            cps = []
            for j in range(NG):
                cp = (pltpu.async_copy if fire else pltpu.make_async_copy)(
                    p_hbm.at[idx_v.at[c % 4].at[j]],
                    rows_v.at[c % 2].at[pl.ds(j * GL, GL)],
                    sems[c % 2],
                )
                cps.append(cp)
            return cps

        idx_copy(0, True).wait()
        idx_copy(1, True)
        gathers(0, True)
        for c in range(NCHUNK):
            buf = c % 2
            if c + 1 < NCHUNK:
                idx_copy(c + 1, False).wait()
                if c + 2 < NCHUNK:
                    idx_copy(c + 2, True)
                gathers(c + 1, True)
            for cp in gathers(c, False):
                cp.wait()

            rbuf = rows_v.at[buf]

            def elem_body(e, _):
                row0 = e * L

                def t_body(i, accs):
                    a0, a1 = accs
                    r = row0 + i * 20
                    for j in range(5):
                        a0 = a0 + plsc.load_gather(
                            rbuf, [rowpat + (r + 4 * j), colpat])
                        a1 = a1 + plsc.load_gather(
                            rbuf, [rowpat + (r + 4 * j + 2), colpat])
                    return (a0, a1)

                z = jnp.zeros((LANES,), jnp.float32)
                a0, a1 = lax.fori_loop(0, L // 20, t_body, (z, z))
                out_v[c * CE + e, :] = a0 + a1
                return 0

            lax.fori_loop(0, CE, elem_body, 0)

        pltpu.sync_copy(out_v, out_hbm.at[pl.ds(wid * EPW, EPW)])

    return k(x2, proj)


@jax.jit
def _impl(x, table, W, b):
    xi = x.astype(jnp.int32)
    xs = (xi & ~(BN - 1)) | ((xi & (SUB - 1)) << 4) | ((xi & (BN - 1)) >> SSH)
    x2 = xs.reshape(B * L // GL, GL)
    w_scaled = (
        jnp.zeros((CH, E), jnp.float32).at[:OUT, :].set(W) * (1.0 / L)
    )
    proj16 = _tc_proj(w_scaled, table.T)
    proj = proj16.reshape(VP, CH)
    out16 = _sc_pool(x2, proj)
    return out16[:, :OUT] + out16[:, CH:CH + OUT] + b[None, :]


def kernel(x, table, W, b):
    return _impl(x, table, W, b)
